# baseline (speedup 1.0000x reference)
import functools

import jax
import jax.numpy as jnp
from jax import lax
from jax.experimental import pallas as pl
from jax.experimental.pallas import tpu as pltpu
from jax.experimental.pallas import tpu_sc as plsc

N = 10000
E = 320000
D_IN = 128
H = 256
HH = H // 2
L = 4
G = 64
NS = 16
W = 64
CH = 40
NCHT = E // W // CH
NPAIR = CH // 2
RB = 8
NCH = N // RB
ET = 104


def _sc_agg_body(h2, emb3, src2r, typ2r, dst2r, out,
                 acc, emb_s, gb0, gb1, eb0, eb1, zbuf, srcc, typc, dstc,
                 sg0, sg1, se0, se1):
    c = lax.axis_index("c")
    s = lax.axis_index("s")

    @pl.when(s < ET // RB)
    def _():
        pltpu.sync_copy(emb3.at[c, pl.ds(s * RB, RB)], zbuf)
        pltpu.sync_copy(zbuf, emb_s.at[pl.ds(s * RB, RB)])

    def _zb(i, carry):
        r = i // 8
        j = i % 8
        zbuf[r, pl.ds(j * 16, 16)] = jnp.zeros((16,), jnp.float32)
        return carry
    lax.fori_loop(0, RB * 8, _zb, 0)
    n_my = (NCH + NS - 1 - s) // NS

    def _zero(k, carry):
        chunk = s + NS * k
        pltpu.sync_copy(zbuf, acc.at[pl.ds(chunk * RB, RB)])
        return carry
    lax.fori_loop(0, n_my, _zero, 0)
    plsc.subcore_barrier()

    def _compute(gb, eb):
        def _comp(k2, carry2):
            for j in range(HH // 16):
                sl = pl.ds(j * 16, 16)
                v = gb[k2, sl] + eb[k2, sl]
                gb[k2, sl] = jnp.maximum(v, 0.0)
            return carry2
        lax.fori_loop(0, W, _comp, 0, unroll=2)

    def _start(w, gb, eb, sg, se):
        pltpu.async_copy(h2.at[srcc.at[w]], gb, sg)
        pltpu.async_copy(emb_s.at[typc.at[w]], eb, se)

    def _finish(w, gb, eb, sg, se):
        pltpu.make_async_copy(h2.at[srcc.at[w]], gb, sg).wait()
        pltpu.make_async_copy(emb_s.at[typc.at[w]], eb, se).wait()
        _compute(gb, eb)
        pltpu.sync_copy(gb, acc.at[dstc.at[w]], add=True)

    n_ch = (NCHT + NS - 1 - s) // NS

    def _chunk(ci, carry):
        r0 = (s + NS * ci) * CH
        pltpu.sync_copy(src2r.at[c, pl.ds(r0, CH)], srcc)
        pltpu.sync_copy(typ2r.at[pl.ds(r0, CH)], typc)
        pltpu.sync_copy(dst2r.at[pl.ds(r0, CH)], dstc)
        _start(0, gb0, eb0, sg0, se0)

        def _pair(j, carry2):
            w0 = 2 * j
            _start(w0 + 1, gb1, eb1, sg1, se1)
            _finish(w0, gb0, eb0, sg0, se0)

            @pl.when(j < NPAIR - 1)
            def _():
                _start(w0 + 2, gb0, eb0, sg0, se0)
            _finish(w0 + 1, gb1, eb1, sg1, se1)
            return carry2
        lax.fori_loop(0, NPAIR, _pair, 0)
        return carry
    lax.fori_loop(0, n_ch, _chunk, 0)
    plsc.subcore_barrier()

    def _writeout(k, carry):
        chunk = s + NS * k
        pltpu.sync_copy(acc.at[pl.ds(chunk * RB, RB)], zbuf)
        pltpu.sync_copy(zbuf, out.at[c, pl.ds(chunk * RB, RB)])
        return carry
    lax.fori_loop(0, n_my, _writeout, 0)


_sc_agg = pl.kernel(
    _sc_agg_body,
    out_type=jax.ShapeDtypeStruct((2, N, HH), jnp.float32),
    mesh=plsc.VectorSubcoreMesh(core_axis_name="c", subcore_axis_name="s"),
    scratch_types=[
        pltpu.VMEM_SHARED((N, HH), jnp.float32),
        pltpu.VMEM_SHARED((ET, HH), jnp.float32),
        pltpu.VMEM((W, HH), jnp.float32),
        pltpu.VMEM((W, HH), jnp.float32),
        pltpu.VMEM((W, HH), jnp.float32),
        pltpu.VMEM((W, HH), jnp.float32),
        pltpu.VMEM((RB, HH), jnp.float32),
        pltpu.VMEM((CH, W), jnp.int32),
        pltpu.VMEM((CH, W), jnp.int32),
        pltpu.VMEM((CH, W), jnp.int32),
        pltpu.SemaphoreType.DMA,
        pltpu.SemaphoreType.DMA,
        pltpu.SemaphoreType.DMA,
        pltpu.SemaphoreType.DMA,
    ],
)

BN = 1000
NB = N // BN


def _in_body(x_ref, w_ref, b_ref, o_ref):
    o_ref[...] = (
        jnp.dot(x_ref[...], w_ref[...], preferred_element_type=jnp.float32,
                precision=lax.Precision.HIGHEST)
        + b_ref[...]
    )


def _input_layer(node, W_in, b_in):
    return pl.pallas_call(
        _in_body,
        grid=(NB,),
        in_specs=[
            pl.BlockSpec((BN, D_IN), lambda i: (i, 0)),
            pl.BlockSpec((D_IN, H), lambda i: (0, 0)),
            pl.BlockSpec((1, H), lambda i: (0, 0)),
        ],
        out_specs=pl.BlockSpec((BN, H), lambda i: (i, 0)),
        out_shape=jax.ShapeDtypeStruct((N, H), jnp.float32),
    )(node, W_in, b_in.reshape(1, H))


def _upd_body(h_ref, a_ref, w_ref, b_ref, o_ref):
    h = h_ref[...]
    x = h + jnp.concatenate([a_ref[0], a_ref[1]], axis=1)
    z = jnp.dot(x, w_ref[...], preferred_element_type=jnp.float32,
                precision=lax.Precision.HIGHEST) + b_ref[...]
    z = jnp.where(z > 0, z, 0.01 * z)
    o_ref[...] = z + h


def _update_layer(h, agg3, Wc_i, bc_i):
    return pl.pallas_call(
        _upd_body,
        grid=(NB,),
        in_specs=[
            pl.BlockSpec((BN, H), lambda i: (i, 0)),
            pl.BlockSpec((2, BN, HH), lambda i: (0, i, 0)),
            pl.BlockSpec((H, H), lambda i: (0, 0)),
            pl.BlockSpec((1, H), lambda i: (0, 0)),
        ],
        out_specs=pl.BlockSpec((BN, H), lambda i: (i, 0)),
        out_shape=jax.ShapeDtypeStruct((N, H), jnp.float32),
    )(h, agg3, Wc_i, bc_i.reshape(1, H))


def _pool_body(h_ref, bi_ref, w_ref, b_ref, o_ref, sums, cnts):
    i = pl.program_id(0)

    @pl.when(i == 0)
    def _():
        sums[...] = jnp.zeros_like(sums)
        cnts[...] = jnp.zeros_like(cnts)

    ids = bi_ref[0, 0, :]
    oh = (ids[:, None] == lax.broadcasted_iota(jnp.int32, (BN, G), 1)).astype(
        jnp.float32)
    sums[...] += lax.dot_general(
        oh, h_ref[...], (((0,), (0,)), ((), ())),
        preferred_element_type=jnp.float32,
        precision=lax.Precision.HIGHEST)
    cnts[...] += jnp.sum(oh, axis=0)[None, :]

    @pl.when(i == NB - 1)
    def _():
        mean = sums[...] / jnp.maximum(cnts[0, :], 1.0)[:, None]
        o_ref[...] = (jnp.sum(mean * w_ref[...], axis=1) + b_ref[0, 0])[None, :]


def _pool_head(h, batch_index, W_out, b_out):
    return pl.pallas_call(
        _pool_body,
        grid=(NB,),
        in_specs=[
            pl.BlockSpec((BN, H), lambda i: (i, 0)),
            pl.BlockSpec((1, 1, BN), lambda i: (i, 0, 0)),
            pl.BlockSpec((1, H), lambda i: (0, 0)),
            pl.BlockSpec((1, 1), lambda i: (0, 0)),
        ],
        out_specs=pl.BlockSpec((1, G), lambda i: (0, 0)),
        out_shape=jax.ShapeDtypeStruct((1, G), jnp.float32),
        scratch_shapes=[
            pltpu.VMEM((G, H), jnp.float32),
            pltpu.VMEM((1, G), jnp.float32),
        ],
    )(h, batch_index.reshape(NB, 1, BN), W_out.reshape(1, H),
      b_out.reshape(1, 1))


def kernel(node, edge, edge_type, batch_index, W_in, b_in, emb, Wc, bc,
           W_out, b_out):
    src1 = edge[:, 0] * 2
    src2 = jnp.stack([src1, src1 + 1]).reshape(2, E // W, W)
    typ2 = edge_type[:, 0].reshape(E // W, W)
    dst2 = edge[:, 1].reshape(E // W, W)
    nt = emb.shape[0]
    emb3 = jnp.pad(emb.reshape(nt, 2, HH).transpose(1, 0, 2),
                   ((0, 0), (0, ET - nt), (0, 0)))
    h = _input_layer(node, W_in, b_in)
    for i in range(L):
        agg3 = _sc_agg(h.reshape(2 * N, HH), emb3, src2, typ2, dst2)
        h = _update_layer(h, agg3, Wc[i], bc[i])
    out = _pool_head(h, batch_index, W_out, b_out)
    return out.reshape(G)


# speedup vs baseline: 1.0088x; 1.0088x over previous
import functools

import jax
import jax.numpy as jnp
from jax import lax
from jax.experimental import pallas as pl
from jax.experimental.pallas import tpu as pltpu
from jax.experimental.pallas import tpu_sc as plsc

N = 10000
E = 320000
D_IN = 128
H = 256
HH = H // 2
L = 4
G = 64
NS = 16
W = 64
CH = 40
NCHT = E // W // CH
NPAIR = CH // 2
RB = 16
NCH = N // RB


def _sc_agg_body(h2, emb2, src2r, typ2r, dst2r, out,
                 acc, gb0, gb1, eb0, eb1, zbuf, srcc, typc, dstc,
                 sg0, sg1, se0, se1):
    c = lax.axis_index("c")
    s = lax.axis_index("s")

    def _zb(i, carry):
        r = i // 8
        j = i % 8
        zbuf[r, pl.ds(j * 16, 16)] = jnp.zeros((16,), jnp.float32)
        return carry
    lax.fori_loop(0, RB * 8, _zb, 0)
    n_my = (NCH + NS - 1 - s) // NS

    def _zero(k, carry):
        chunk = s + NS * k
        pltpu.sync_copy(zbuf, acc.at[pl.ds(chunk * RB, RB)])
        return carry
    lax.fori_loop(0, n_my, _zero, 0)
    plsc.subcore_barrier()

    def _compute(gb, eb):
        def _comp(k2, carry2):
            for j in range(HH // 16):
                sl = pl.ds(j * 16, 16)
                v = gb[k2, sl] + eb[k2, sl]
                gb[k2, sl] = jnp.maximum(v, 0.0)
            return carry2
        lax.fori_loop(0, W, _comp, 0, unroll=2)

    def _start(w, gb, eb, sg, se):
        pltpu.async_copy(h2.at[srcc.at[w]], gb, sg)
        pltpu.async_copy(emb2.at[typc.at[w]], eb, se)

    def _finish(w, gb, eb, sg, se):
        pltpu.make_async_copy(h2.at[srcc.at[w]], gb, sg).wait()
        pltpu.make_async_copy(emb2.at[typc.at[w]], eb, se).wait()
        _compute(gb, eb)
        pltpu.sync_copy(gb, acc.at[dstc.at[w]], add=True)

    n_ch = (NCHT + NS - 1 - s) // NS

    def _chunk(ci, carry):
        r0 = (s + NS * ci) * CH
        pltpu.sync_copy(src2r.at[c, pl.ds(r0, CH)], srcc)
        pltpu.sync_copy(typ2r.at[c, pl.ds(r0, CH)], typc)
        pltpu.sync_copy(dst2r.at[pl.ds(r0, CH)], dstc)
        _start(0, gb0, eb0, sg0, se0)

        def _pair(j, carry2):
            w0 = 2 * j
            _start(w0 + 1, gb1, eb1, sg1, se1)
            _finish(w0, gb0, eb0, sg0, se0)

            @pl.when(j < NPAIR - 1)
            def _():
                _start(w0 + 2, gb0, eb0, sg0, se0)
            _finish(w0 + 1, gb1, eb1, sg1, se1)
            return carry2
        lax.fori_loop(0, NPAIR, _pair, 0)
        return carry
    lax.fori_loop(0, n_ch, _chunk, 0)
    plsc.subcore_barrier()

    def _writeout(k, carry):
        chunk = s + NS * k
        pltpu.sync_copy(acc.at[pl.ds(chunk * RB, RB)], zbuf)
        pltpu.sync_copy(zbuf, out.at[c, pl.ds(chunk * RB, RB)])
        return carry
    lax.fori_loop(0, n_my, _writeout, 0)


_sc_agg = pl.kernel(
    _sc_agg_body,
    out_type=jax.ShapeDtypeStruct((2, N, HH), jnp.float32),
    mesh=plsc.VectorSubcoreMesh(core_axis_name="c", subcore_axis_name="s"),
    scratch_types=[
        pltpu.VMEM_SHARED((N, HH), jnp.float32),
        pltpu.VMEM((W, HH), jnp.float32),
        pltpu.VMEM((W, HH), jnp.float32),
        pltpu.VMEM((W, HH), jnp.float32),
        pltpu.VMEM((W, HH), jnp.float32),
        pltpu.VMEM((RB, HH), jnp.float32),
        pltpu.VMEM((CH, W), jnp.int32),
        pltpu.VMEM((CH, W), jnp.int32),
        pltpu.VMEM((CH, W), jnp.int32),
        pltpu.SemaphoreType.DMA,
        pltpu.SemaphoreType.DMA,
        pltpu.SemaphoreType.DMA,
        pltpu.SemaphoreType.DMA,
    ],
)

BN = 1000
NB = N // BN


def _in_body(x_ref, w_ref, b_ref, o_ref):
    o_ref[...] = (
        jnp.dot(x_ref[...], w_ref[...], preferred_element_type=jnp.float32,
                precision=lax.Precision.HIGHEST)
        + b_ref[...]
    )


def _input_layer(node, W_in, b_in):
    return pl.pallas_call(
        _in_body,
        grid=(NB,),
        in_specs=[
            pl.BlockSpec((BN, D_IN), lambda i: (i, 0)),
            pl.BlockSpec((D_IN, H), lambda i: (0, 0)),
            pl.BlockSpec((1, H), lambda i: (0, 0)),
        ],
        out_specs=pl.BlockSpec((BN, H), lambda i: (i, 0)),
        out_shape=jax.ShapeDtypeStruct((N, H), jnp.float32),
    )(node, W_in, b_in.reshape(1, H))


def _upd_body(h_ref, a_ref, w_ref, b_ref, o_ref):
    h = h_ref[...]
    x = h + jnp.concatenate([a_ref[0], a_ref[1]], axis=1)
    z = jnp.dot(x, w_ref[...], preferred_element_type=jnp.float32,
                precision=lax.Precision.HIGHEST) + b_ref[...]
    z = jnp.where(z > 0, z, 0.01 * z)
    o_ref[...] = z + h


def _update_layer(h, agg3, Wc_i, bc_i):
    return pl.pallas_call(
        _upd_body,
        grid=(NB,),
        in_specs=[
            pl.BlockSpec((BN, H), lambda i: (i, 0)),
            pl.BlockSpec((2, BN, HH), lambda i: (0, i, 0)),
            pl.BlockSpec((H, H), lambda i: (0, 0)),
            pl.BlockSpec((1, H), lambda i: (0, 0)),
        ],
        out_specs=pl.BlockSpec((BN, H), lambda i: (i, 0)),
        out_shape=jax.ShapeDtypeStruct((N, H), jnp.float32),
    )(h, agg3, Wc_i, bc_i.reshape(1, H))


def _pool_body(h_ref, bi_ref, w_ref, b_ref, o_ref, sums, cnts):
    i = pl.program_id(0)

    @pl.when(i == 0)
    def _():
        sums[...] = jnp.zeros_like(sums)
        cnts[...] = jnp.zeros_like(cnts)

    ids = bi_ref[0, 0, :]
    oh = (ids[:, None] == lax.broadcasted_iota(jnp.int32, (BN, G), 1)).astype(
        jnp.float32)
    sums[...] += lax.dot_general(
        oh, h_ref[...], (((0,), (0,)), ((), ())),
        preferred_element_type=jnp.float32,
        precision=lax.Precision.HIGHEST)
    cnts[...] += jnp.sum(oh, axis=0)[None, :]

    @pl.when(i == NB - 1)
    def _():
        mean = sums[...] / jnp.maximum(cnts[0, :], 1.0)[:, None]
        o_ref[...] = (jnp.sum(mean * w_ref[...], axis=1) + b_ref[0, 0])[None, :]


def _pool_head(h, batch_index, W_out, b_out):
    return pl.pallas_call(
        _pool_body,
        grid=(NB,),
        in_specs=[
            pl.BlockSpec((BN, H), lambda i: (i, 0)),
            pl.BlockSpec((1, 1, BN), lambda i: (i, 0, 0)),
            pl.BlockSpec((1, H), lambda i: (0, 0)),
            pl.BlockSpec((1, 1), lambda i: (0, 0)),
        ],
        out_specs=pl.BlockSpec((1, G), lambda i: (0, 0)),
        out_shape=jax.ShapeDtypeStruct((1, G), jnp.float32),
        scratch_shapes=[
            pltpu.VMEM((G, H), jnp.float32),
            pltpu.VMEM((1, G), jnp.float32),
        ],
    )(h, batch_index.reshape(NB, 1, BN), W_out.reshape(1, H),
      b_out.reshape(1, 1))


def kernel(node, edge, edge_type, batch_index, W_in, b_in, emb, Wc, bc,
           W_out, b_out):
    src1 = edge[:, 0] * 2
    typ1 = edge_type[:, 0] * 2
    src2 = jnp.stack([src1, src1 + 1]).reshape(2, E // W, W)
    typ2 = jnp.stack([typ1, typ1 + 1]).reshape(2, E // W, W)
    dst2 = edge[:, 1].reshape(E // W, W)
    emb2 = emb.reshape(2 * emb.shape[0], HH)
    h = _input_layer(node, W_in, b_in)
    for i in range(L):
        agg3 = _sc_agg(h.reshape(2 * N, HH), emb2, src2, typ2, dst2)
        h = _update_layer(h, agg3, Wc[i], bc[i])
    out = _pool_head(h, batch_index, W_out, b_out)
    return out.reshape(G)


# speedup vs baseline: 1.8888x; 1.8724x over previous
import functools

import jax
import jax.numpy as jnp
from jax import lax
from jax.experimental import pallas as pl
from jax.experimental.pallas import tpu as pltpu
from jax.experimental.pallas import tpu_sc as plsc

N = 10000
E = 320000
D_IN = 128
H = 256
HH = H // 2
L = 4
G = 64
NS = 16
W = 64
CH = 40
NCHT = E // W // CH
NPAIR = CH // 2
RB = 16
NCH = N // RB


def _sc_agg_body(h2, emb2, src2r, typ2r, dst2r, out,
                 acc, gb0, gb1, eb0, eb1, zbuf, srcc, typc, dstc,
                 sg0, sg1, se0, se1, ss0, ss1):
    c = lax.axis_index("c")
    s = lax.axis_index("s")

    def _zb(i, carry):
        r = i // 8
        j = i % 8
        zbuf[r, pl.ds(j * 16, 16)] = jnp.zeros((16,), jnp.float32)
        return carry
    lax.fori_loop(0, RB * 8, _zb, 0)
    n_my = (NCH + NS - 1 - s) // NS

    def _zero(k, carry):
        chunk = s + NS * k
        pltpu.sync_copy(zbuf, acc.at[pl.ds(chunk * RB, RB)])
        return carry
    lax.fori_loop(0, n_my, _zero, 0)
    plsc.subcore_barrier()

    def _compute(gb, eb):
        def _comp(k2, carry2):
            for j in range(HH // 16):
                sl = pl.ds(j * 16, 16)
                v = gb[k2, sl] + eb[k2, sl]
                gb[k2, sl] = jnp.maximum(v, 0.0)
            return carry2
        lax.fori_loop(0, W, _comp, 0)

    def _start(w, gb, eb, sg, se):
        pltpu.async_copy(h2.at[srcc.at[w]], gb, sg)
        pltpu.async_copy(emb2.at[typc.at[w]], eb, se)

    def _finish(w, gb, eb, sg, se, ss):
        pltpu.make_async_copy(h2.at[srcc.at[w]], gb, sg).wait()
        pltpu.make_async_copy(emb2.at[typc.at[w]], eb, se).wait()
        _compute(gb, eb)
        pltpu.async_copy(gb, acc.at[dstc.at[w]], ss, add=True)

    def _wait_scatter(w, gb, ss):
        pltpu.make_async_copy(gb, acc.at[dstc.at[w]], ss).wait()

    n_ch = (NCHT + NS - 1 - s) // NS

    def _chunk(ci, carry):
        r0 = (s + NS * ci) * CH
        pltpu.sync_copy(src2r.at[c, pl.ds(r0, CH)], srcc)
        pltpu.sync_copy(typ2r.at[c, pl.ds(r0, CH)], typc)
        pltpu.sync_copy(dst2r.at[pl.ds(r0, CH)], dstc)
        _start(0, gb0, eb0, sg0, se0)

        def _pair(j, carry2):
            w0 = 2 * j

            @pl.when(j > 0)
            def _():
                _wait_scatter(w0 - 1, gb1, ss1)
            _start(w0 + 1, gb1, eb1, sg1, se1)
            _finish(w0, gb0, eb0, sg0, se0, ss0)
            _finish(w0 + 1, gb1, eb1, sg1, se1, ss1)

            @pl.when(j < NPAIR - 1)
            def _():
                _wait_scatter(w0, gb0, ss0)
                _start(w0 + 2, gb0, eb0, sg0, se0)
            return carry2
        lax.fori_loop(0, NPAIR, _pair, 0)
        _wait_scatter(CH - 2, gb0, ss0)
        _wait_scatter(CH - 1, gb1, ss1)
        return carry
    lax.fori_loop(0, n_ch, _chunk, 0)
    plsc.subcore_barrier()

    def _writeout(k, carry):
        chunk = s + NS * k
        pltpu.sync_copy(acc.at[pl.ds(chunk * RB, RB)], zbuf)
        pltpu.sync_copy(zbuf, out.at[c, pl.ds(chunk * RB, RB)])
        return carry
    lax.fori_loop(0, n_my, _writeout, 0)


_sc_agg = pl.kernel(
    _sc_agg_body,
    out_type=jax.ShapeDtypeStruct((2, N, HH), jnp.float32),
    mesh=plsc.VectorSubcoreMesh(core_axis_name="c", subcore_axis_name="s"),
    scratch_types=[
        pltpu.VMEM_SHARED((N, HH), jnp.float32),
        pltpu.VMEM((W, HH), jnp.float32),
        pltpu.VMEM((W, HH), jnp.float32),
        pltpu.VMEM((W, HH), jnp.float32),
        pltpu.VMEM((W, HH), jnp.float32),
        pltpu.VMEM((RB, HH), jnp.float32),
        pltpu.VMEM((CH, W), jnp.int32),
        pltpu.VMEM((CH, W), jnp.int32),
        pltpu.VMEM((CH, W), jnp.int32),
        pltpu.SemaphoreType.DMA,
        pltpu.SemaphoreType.DMA,
        pltpu.SemaphoreType.DMA,
        pltpu.SemaphoreType.DMA,
        pltpu.SemaphoreType.DMA,
        pltpu.SemaphoreType.DMA,
    ],
)

BN = 1000
NB = N // BN


def _in_body(x_ref, w_ref, b_ref, o_ref):
    o_ref[...] = (
        jnp.dot(x_ref[...], w_ref[...], preferred_element_type=jnp.float32,
                precision=lax.Precision.HIGHEST)
        + b_ref[...]
    )


def _input_layer(node, W_in, b_in):
    return pl.pallas_call(
        _in_body,
        grid=(NB,),
        in_specs=[
            pl.BlockSpec((BN, D_IN), lambda i: (i, 0)),
            pl.BlockSpec((D_IN, H), lambda i: (0, 0)),
            pl.BlockSpec((1, H), lambda i: (0, 0)),
        ],
        out_specs=pl.BlockSpec((BN, H), lambda i: (i, 0)),
        out_shape=jax.ShapeDtypeStruct((N, H), jnp.float32),
    )(node, W_in, b_in.reshape(1, H))


def _upd_body(h_ref, a_ref, w_ref, b_ref, o_ref):
    h = h_ref[...]
    x = h + jnp.concatenate([a_ref[0], a_ref[1]], axis=1)
    z = jnp.dot(x, w_ref[...], preferred_element_type=jnp.float32,
                precision=lax.Precision.HIGHEST) + b_ref[...]
    z = jnp.where(z > 0, z, 0.01 * z)
    o_ref[...] = z + h


def _update_layer(h, agg3, Wc_i, bc_i):
    return pl.pallas_call(
        _upd_body,
        grid=(NB,),
        in_specs=[
            pl.BlockSpec((BN, H), lambda i: (i, 0)),
            pl.BlockSpec((2, BN, HH), lambda i: (0, i, 0)),
            pl.BlockSpec((H, H), lambda i: (0, 0)),
            pl.BlockSpec((1, H), lambda i: (0, 0)),
        ],
        out_specs=pl.BlockSpec((BN, H), lambda i: (i, 0)),
        out_shape=jax.ShapeDtypeStruct((N, H), jnp.float32),
    )(h, agg3, Wc_i, bc_i.reshape(1, H))


def _pool_body(h_ref, bi_ref, w_ref, b_ref, o_ref, sums, cnts):
    i = pl.program_id(0)

    @pl.when(i == 0)
    def _():
        sums[...] = jnp.zeros_like(sums)
        cnts[...] = jnp.zeros_like(cnts)

    ids = bi_ref[0, 0, :]
    oh = (ids[:, None] == lax.broadcasted_iota(jnp.int32, (BN, G), 1)).astype(
        jnp.float32)
    sums[...] += lax.dot_general(
        oh, h_ref[...], (((0,), (0,)), ((), ())),
        preferred_element_type=jnp.float32,
        precision=lax.Precision.HIGHEST)
    cnts[...] += jnp.sum(oh, axis=0)[None, :]

    @pl.when(i == NB - 1)
    def _():
        mean = sums[...] / jnp.maximum(cnts[0, :], 1.0)[:, None]
        o_ref[...] = (jnp.sum(mean * w_ref[...], axis=1) + b_ref[0, 0])[None, :]


def _pool_head(h, batch_index, W_out, b_out):
    return pl.pallas_call(
        _pool_body,
        grid=(NB,),
        in_specs=[
            pl.BlockSpec((BN, H), lambda i: (i, 0)),
            pl.BlockSpec((1, 1, BN), lambda i: (i, 0, 0)),
            pl.BlockSpec((1, H), lambda i: (0, 0)),
            pl.BlockSpec((1, 1), lambda i: (0, 0)),
        ],
        out_specs=pl.BlockSpec((1, G), lambda i: (0, 0)),
        out_shape=jax.ShapeDtypeStruct((1, G), jnp.float32),
        scratch_shapes=[
            pltpu.VMEM((G, H), jnp.float32),
            pltpu.VMEM((1, G), jnp.float32),
        ],
    )(h, batch_index.reshape(NB, 1, BN), W_out.reshape(1, H),
      b_out.reshape(1, 1))


def kernel(node, edge, edge_type, batch_index, W_in, b_in, emb, Wc, bc,
           W_out, b_out):
    src1 = edge[:, 0] * 2
    typ1 = edge_type[:, 0] * 2
    src2 = jnp.stack([src1, src1 + 1]).reshape(2, E // W, W)
    typ2 = jnp.stack([typ1, typ1 + 1]).reshape(2, E // W, W)
    dst2 = edge[:, 1].reshape(E // W, W)
    emb2 = emb.reshape(2 * emb.shape[0], HH)
    h = _input_layer(node, W_in, b_in)
    for i in range(L):
        agg3 = _sc_agg(h.reshape(2 * N, HH), emb2, src2, typ2, dst2)
        h = _update_layer(h, agg3, Wc[i], bc[i])
    out = _pool_head(h, batch_index, W_out, b_out)
    return out.reshape(G)


# speedup vs baseline: 1.9661x; 1.0409x over previous
import functools

import jax
import jax.numpy as jnp
from jax import lax
from jax.experimental import pallas as pl
from jax.experimental.pallas import tpu as pltpu
from jax.experimental.pallas import tpu_sc as plsc

N = 10000
E = 320000
D_IN = 128
H = 256
HH = H // 2
L = 4
G = 64
NS = 16
W = 64
CH = 40
NCHT = E // W // CH
NPAIR = CH // 2
RB = 16
NCH = N // RB


def _sc_agg_body(h2, emb2, src2r, typ2r, dst2r, out,
                 acc, gb0, gb1, eb0, eb1, zbuf, srcc, typc, dstc,
                 sg0, sg1, se0, se1):
    c = lax.axis_index("c")
    s = lax.axis_index("s")

    def _zb(i, carry):
        r = i // 8
        j = i % 8
        zbuf[r, pl.ds(j * 16, 16)] = jnp.zeros((16,), jnp.float32)
        return carry
    lax.fori_loop(0, RB * 8, _zb, 0)
    n_my = (NCH + NS - 1 - s) // NS

    def _zero(k, carry):
        chunk = s + NS * k
        pltpu.sync_copy(zbuf, acc.at[pl.ds(chunk * RB, RB)])
        return carry
    lax.fori_loop(0, n_my, _zero, 0)
    plsc.subcore_barrier()

    def _compute(gb, eb):
        def _comp(k2, carry2):
            for j in range(HH // 16):
                sl = pl.ds(j * 16, 16)
                v = gb[k2, sl] + eb[k2, sl]
                gb[k2, sl] = jnp.maximum(v, 0.0)
            return carry2
        lax.fori_loop(0, W, _comp, 0)

    def _start(w, gb, eb, sg, se):
        pltpu.async_copy(h2.at[srcc.at[w]], gb, sg)
        pltpu.async_copy(emb2.at[typc.at[w]], eb, se)

    def _finish(w, gb, eb, sg, se):
        pltpu.make_async_copy(h2.at[srcc.at[w]], gb, sg).wait()
        pltpu.make_async_copy(emb2.at[typc.at[w]], eb, se).wait()
        _compute(gb, eb)
        pltpu.sync_copy(gb, acc.at[dstc.at[w]], add=True)

    n_ch = (NCHT + NS - 1 - s) // NS

    def _chunk(ci, carry):
        r0 = (s + NS * ci) * CH
        pltpu.sync_copy(src2r.at[c, pl.ds(r0, CH)], srcc)
        pltpu.sync_copy(typ2r.at[c, pl.ds(r0, CH)], typc)
        pltpu.sync_copy(dst2r.at[pl.ds(r0, CH)], dstc)
        _start(0, gb0, eb0, sg0, se0)

        def _pair(j, carry2):
            w0 = 2 * j
            _start(w0 + 1, gb1, eb1, sg1, se1)
            _finish(w0, gb0, eb0, sg0, se0)

            @pl.when(j < NPAIR - 1)
            def _():
                _start(w0 + 2, gb0, eb0, sg0, se0)
            _finish(w0 + 1, gb1, eb1, sg1, se1)
            return carry2
        lax.fori_loop(0, NPAIR, _pair, 0)
        return carry
    lax.fori_loop(0, n_ch, _chunk, 0)
    plsc.subcore_barrier()

    def _writeout(k, carry):
        chunk = s + NS * k
        pltpu.sync_copy(acc.at[pl.ds(chunk * RB, RB)], zbuf)
        pltpu.sync_copy(zbuf, out.at[c, pl.ds(chunk * RB, RB)])
        return carry
    lax.fori_loop(0, n_my, _writeout, 0)


_sc_agg = pl.kernel(
    _sc_agg_body,
    out_type=jax.ShapeDtypeStruct((2, N, HH), jnp.float32),
    mesh=plsc.VectorSubcoreMesh(core_axis_name="c", subcore_axis_name="s"),
    scratch_types=[
        pltpu.VMEM_SHARED((N, HH), jnp.float32),
        pltpu.VMEM((W, HH), jnp.float32),
        pltpu.VMEM((W, HH), jnp.float32),
        pltpu.VMEM((W, HH), jnp.float32),
        pltpu.VMEM((W, HH), jnp.float32),
        pltpu.VMEM((RB, HH), jnp.float32),
        pltpu.VMEM((CH, W), jnp.int32),
        pltpu.VMEM((CH, W), jnp.int32),
        pltpu.VMEM((CH, W), jnp.int32),
        pltpu.SemaphoreType.DMA,
        pltpu.SemaphoreType.DMA,
        pltpu.SemaphoreType.DMA,
        pltpu.SemaphoreType.DMA,
    ],
)

BN = 1000
NB = N // BN


def _in_body(x_ref, w_ref, b_ref, o_ref):
    o_ref[...] = (
        jnp.dot(x_ref[...], w_ref[...], preferred_element_type=jnp.float32,
                precision=lax.Precision.HIGHEST)
        + b_ref[...]
    )


def _input_layer(node, W_in, b_in):
    return pl.pallas_call(
        _in_body,
        grid=(NB,),
        in_specs=[
            pl.BlockSpec((BN, D_IN), lambda i: (i, 0)),
            pl.BlockSpec((D_IN, H), lambda i: (0, 0)),
            pl.BlockSpec((1, H), lambda i: (0, 0)),
        ],
        out_specs=pl.BlockSpec((BN, H), lambda i: (i, 0)),
        out_shape=jax.ShapeDtypeStruct((N, H), jnp.float32),
    )(node, W_in, b_in.reshape(1, H))


def _upd_body(h_ref, a_ref, w_ref, b_ref, o_ref):
    h = h_ref[...]
    x = h + jnp.concatenate([a_ref[0], a_ref[1]], axis=1)
    z = jnp.dot(x, w_ref[...], preferred_element_type=jnp.float32,
                precision=lax.Precision.HIGHEST) + b_ref[...]
    z = jnp.where(z > 0, z, 0.01 * z)
    o_ref[...] = z + h


def _update_layer(h, agg3, Wc_i, bc_i):
    return pl.pallas_call(
        _upd_body,
        grid=(NB,),
        in_specs=[
            pl.BlockSpec((BN, H), lambda i: (i, 0)),
            pl.BlockSpec((2, BN, HH), lambda i: (0, i, 0)),
            pl.BlockSpec((H, H), lambda i: (0, 0)),
            pl.BlockSpec((1, H), lambda i: (0, 0)),
        ],
        out_specs=pl.BlockSpec((BN, H), lambda i: (i, 0)),
        out_shape=jax.ShapeDtypeStruct((N, H), jnp.float32),
    )(h, agg3, Wc_i, bc_i.reshape(1, H))


def _pool_body(h_ref, bi_ref, w_ref, b_ref, o_ref, sums, cnts):
    i = pl.program_id(0)

    @pl.when(i == 0)
    def _():
        sums[...] = jnp.zeros_like(sums)
        cnts[...] = jnp.zeros_like(cnts)

    ids = bi_ref[0, 0, :]
    oh = (ids[:, None] == lax.broadcasted_iota(jnp.int32, (BN, G), 1)).astype(
        jnp.float32)
    sums[...] += lax.dot_general(
        oh, h_ref[...], (((0,), (0,)), ((), ())),
        preferred_element_type=jnp.float32,
        precision=lax.Precision.HIGHEST)
    cnts[...] += jnp.sum(oh, axis=0)[None, :]

    @pl.when(i == NB - 1)
    def _():
        mean = sums[...] / jnp.maximum(cnts[0, :], 1.0)[:, None]
        o_ref[...] = (jnp.sum(mean * w_ref[...], axis=1) + b_ref[0, 0])[None, :]


def _pool_head(h, batch_index, W_out, b_out):
    return pl.pallas_call(
        _pool_body,
        grid=(NB,),
        in_specs=[
            pl.BlockSpec((BN, H), lambda i: (i, 0)),
            pl.BlockSpec((1, 1, BN), lambda i: (i, 0, 0)),
            pl.BlockSpec((1, H), lambda i: (0, 0)),
            pl.BlockSpec((1, 1), lambda i: (0, 0)),
        ],
        out_specs=pl.BlockSpec((1, G), lambda i: (0, 0)),
        out_shape=jax.ShapeDtypeStruct((1, G), jnp.float32),
        scratch_shapes=[
            pltpu.VMEM((G, H), jnp.float32),
            pltpu.VMEM((1, G), jnp.float32),
        ],
    )(h, batch_index.reshape(NB, 1, BN), W_out.reshape(1, H),
      b_out.reshape(1, 1))


def kernel(node, edge, edge_type, batch_index, W_in, b_in, emb, Wc, bc,
           W_out, b_out):
    src1 = edge[:, 0] * 2
    typ1 = edge_type[:, 0] * 2
    src2 = jnp.stack([src1, src1 + 1]).reshape(2, E // W, W)
    typ2 = jnp.stack([typ1, typ1 + 1]).reshape(2, E // W, W)
    dst2 = edge[:, 1].reshape(E // W, W)
    emb2 = emb.reshape(2 * emb.shape[0], HH)
    h = _input_layer(node, W_in, b_in)
    for i in range(L):
        agg3 = _sc_agg(h.reshape(2 * N, HH), emb2, src2, typ2, dst2)
        h = _update_layer(h, agg3, Wc[i], bc[i])
    out = _pool_head(h, batch_index, W_out, b_out)
    return out.reshape(G)


# speedup vs baseline: 1.9767x; 1.0054x over previous
import functools

import jax
import jax.numpy as jnp
from jax import lax
from jax.experimental import pallas as pl
from jax.experimental.pallas import tpu as pltpu
from jax.experimental.pallas import tpu_sc as plsc

N = 10000
E = 320000
D_IN = 128
H = 256
HH = H // 2
L = 4
G = 64
NS = 16
W = 64
CH = 40
NCHT = E // W // CH
NPAIR = CH // 2
RB = 16
NCH = N // RB


def _sc_agg_body(h2, emb2, src2r, typ2r, dst2r, out,
                 acc, gb0, gb1, eb0, eb1, zbuf, srcc, typc, dstc,
                 sg0, sg1, se0, se1):
    c = lax.axis_index("c")
    s = lax.axis_index("s")

    def _zb(i, carry):
        r = i // 8
        j = i % 8
        zbuf[r, pl.ds(j * 16, 16)] = jnp.zeros((16,), jnp.float32)
        return carry
    lax.fori_loop(0, RB * 8, _zb, 0)
    n_my = (NCH + NS - 1 - s) // NS

    def _zero(k, carry):
        chunk = s + NS * k
        pltpu.sync_copy(zbuf, acc.at[pl.ds(chunk * RB, RB)])
        return carry
    lax.fori_loop(0, n_my, _zero, 0)
    plsc.subcore_barrier()

    def _compute(gb, eb):
        def _comp(k2, carry2):
            for j in range(HH // 16):
                sl = pl.ds(j * 16, 16)
                v = gb[k2, sl] + eb[k2, sl]
                gb[k2, sl] = jnp.maximum(v, 0.0)
            return carry2
        lax.fori_loop(0, W, _comp, 0)

    def _start(w, gb, eb, sg, se):
        pltpu.async_copy(h2.at[srcc.at[w]], gb, sg)
        pltpu.async_copy(emb2.at[typc.at[w]], eb, se)

    def _finish(w, gb, eb, sg, se):
        pltpu.make_async_copy(h2.at[srcc.at[w]], gb, sg).wait()
        pltpu.make_async_copy(emb2.at[typc.at[w]], eb, se).wait()
        _compute(gb, eb)
        pltpu.sync_copy(gb, acc.at[dstc.at[w]], add=True)

    n_ch = (NCHT + NS - 1 - s) // NS

    def _chunk(ci, carry):
        r0 = (s + NS * ci) * CH
        pltpu.sync_copy(src2r.at[c, pl.ds(r0, CH)], srcc)
        pltpu.sync_copy(typ2r.at[c, pl.ds(r0, CH)], typc)
        pltpu.sync_copy(dst2r.at[pl.ds(r0, CH)], dstc)
        _start(0, gb0, eb0, sg0, se0)

        def _pair(j, carry2):
            w0 = 2 * j
            _start(w0 + 1, gb1, eb1, sg1, se1)
            _finish(w0, gb0, eb0, sg0, se0)

            @pl.when(j < NPAIR - 1)
            def _():
                _start(w0 + 2, gb0, eb0, sg0, se0)
            _finish(w0 + 1, gb1, eb1, sg1, se1)
            return carry2
        lax.fori_loop(0, NPAIR, _pair, 0)
        return carry
    lax.fori_loop(0, n_ch, _chunk, 0)
    plsc.subcore_barrier()

    def _writeout(k, carry):
        chunk = s + NS * k
        pltpu.sync_copy(acc.at[pl.ds(chunk * RB, RB)], zbuf)
        pltpu.sync_copy(zbuf, out.at[c, pl.ds(chunk * RB, RB)])
        return carry
    lax.fori_loop(0, n_my, _writeout, 0)


_sc_agg = pl.kernel(
    _sc_agg_body,
    out_type=jax.ShapeDtypeStruct((2, N, HH), jnp.float32),
    mesh=plsc.VectorSubcoreMesh(core_axis_name="c", subcore_axis_name="s"),
    scratch_types=[
        pltpu.VMEM_SHARED((N, HH), jnp.float32),
        pltpu.VMEM((W, HH), jnp.float32),
        pltpu.VMEM((W, HH), jnp.float32),
        pltpu.VMEM((W, HH), jnp.float32),
        pltpu.VMEM((W, HH), jnp.float32),
        pltpu.VMEM((RB, HH), jnp.float32),
        pltpu.VMEM((CH, W), jnp.int32),
        pltpu.VMEM((CH, W), jnp.int32),
        pltpu.VMEM((CH, W), jnp.int32),
        pltpu.SemaphoreType.DMA,
        pltpu.SemaphoreType.DMA,
        pltpu.SemaphoreType.DMA,
        pltpu.SemaphoreType.DMA,
    ],
)

BN = 2000
NB = N // BN


def _in_body(x_ref, w_ref, b_ref, o_ref):
    o_ref[...] = (
        jnp.dot(x_ref[...], w_ref[...], preferred_element_type=jnp.float32,
                precision=lax.Precision.HIGHEST)
        + b_ref[...]
    )


def _input_layer(node, W_in, b_in):
    return pl.pallas_call(
        _in_body,
        grid=(NB,),
        in_specs=[
            pl.BlockSpec((BN, D_IN), lambda i: (i, 0)),
            pl.BlockSpec((D_IN, H), lambda i: (0, 0)),
            pl.BlockSpec((1, H), lambda i: (0, 0)),
        ],
        out_specs=pl.BlockSpec((BN, H), lambda i: (i, 0)),
        out_shape=jax.ShapeDtypeStruct((N, H), jnp.float32),
    )(node, W_in, b_in.reshape(1, H))


def _upd_body(h_ref, a_ref, w_ref, b_ref, o_ref):
    h = h_ref[...]
    x = h + jnp.concatenate([a_ref[0], a_ref[1]], axis=1)
    z = jnp.dot(x, w_ref[...], preferred_element_type=jnp.float32,
                precision=lax.Precision.HIGHEST) + b_ref[...]
    z = jnp.where(z > 0, z, 0.01 * z)
    o_ref[...] = z + h


def _update_layer(h, agg3, Wc_i, bc_i):
    return pl.pallas_call(
        _upd_body,
        grid=(NB,),
        in_specs=[
            pl.BlockSpec((BN, H), lambda i: (i, 0)),
            pl.BlockSpec((2, BN, HH), lambda i: (0, i, 0)),
            pl.BlockSpec((H, H), lambda i: (0, 0)),
            pl.BlockSpec((1, H), lambda i: (0, 0)),
        ],
        out_specs=pl.BlockSpec((BN, H), lambda i: (i, 0)),
        out_shape=jax.ShapeDtypeStruct((N, H), jnp.float32),
    )(h, agg3, Wc_i, bc_i.reshape(1, H))


def _pool_body(h_ref, bi_ref, w_ref, b_ref, o_ref, sums, cnts):
    i = pl.program_id(0)

    @pl.when(i == 0)
    def _():
        sums[...] = jnp.zeros_like(sums)
        cnts[...] = jnp.zeros_like(cnts)

    ids = bi_ref[0, 0, :]
    oh = (ids[:, None] == lax.broadcasted_iota(jnp.int32, (BN, G), 1)).astype(
        jnp.float32)
    sums[...] += lax.dot_general(
        oh, h_ref[...], (((0,), (0,)), ((), ())),
        preferred_element_type=jnp.float32,
        precision=lax.Precision.HIGHEST)
    cnts[...] += jnp.sum(oh, axis=0)[None, :]

    @pl.when(i == NB - 1)
    def _():
        mean = sums[...] / jnp.maximum(cnts[0, :], 1.0)[:, None]
        o_ref[...] = (jnp.sum(mean * w_ref[...], axis=1) + b_ref[0, 0])[None, :]


def _pool_head(h, batch_index, W_out, b_out):
    return pl.pallas_call(
        _pool_body,
        grid=(NB,),
        in_specs=[
            pl.BlockSpec((BN, H), lambda i: (i, 0)),
            pl.BlockSpec((1, 1, BN), lambda i: (i, 0, 0)),
            pl.BlockSpec((1, H), lambda i: (0, 0)),
            pl.BlockSpec((1, 1), lambda i: (0, 0)),
        ],
        out_specs=pl.BlockSpec((1, G), lambda i: (0, 0)),
        out_shape=jax.ShapeDtypeStruct((1, G), jnp.float32),
        scratch_shapes=[
            pltpu.VMEM((G, H), jnp.float32),
            pltpu.VMEM((1, G), jnp.float32),
        ],
    )(h, batch_index.reshape(NB, 1, BN), W_out.reshape(1, H),
      b_out.reshape(1, 1))


def kernel(node, edge, edge_type, batch_index, W_in, b_in, emb, Wc, bc,
           W_out, b_out):
    src1 = edge[:, 0] * 2
    typ1 = edge_type[:, 0] * 2
    src2 = jnp.stack([src1, src1 + 1]).reshape(2, E // W, W)
    typ2 = jnp.stack([typ1, typ1 + 1]).reshape(2, E // W, W)
    dst2 = edge[:, 1].reshape(E // W, W)
    emb2 = emb.reshape(2 * emb.shape[0], HH)
    h = _input_layer(node, W_in, b_in)
    for i in range(L):
        agg3 = _sc_agg(h.reshape(2 * N, HH), emb2, src2, typ2, dst2)
        h = _update_layer(h, agg3, Wc[i], bc[i])
    out = _pool_head(h, batch_index, W_out, b_out)
    return out.reshape(G)
